# baseline (speedup 1.0000x reference)
import functools

import jax
import jax.numpy as jnp
from jax import lax
from jax.experimental import pallas as pl
from jax.experimental.pallas import tpu as pltpu
from jax.experimental.pallas import tpu_sc as plsc

_B_ROWS = 16384
_SEQ = 20
_DIM = 64
_N_IDX = _B_ROWS * _SEQ

_NC = 2
_NS = 16
_NW = _NC * _NS

_IDXW = 128
_ROWS_PER_W = _N_IDX // _NW
_IDX_ROWS_PER_W = _ROWS_PER_W // _IDXW

_GATHERS_PER_CHUNK = 4
_CHUNK = _GATHERS_PER_CHUNK * _IDXW
_N_CHUNKS = _ROWS_PER_W // _CHUNK


def _make_gather():
  mesh = plsc.VectorSubcoreMesh(core_axis_name="c", subcore_axis_name="s")

  @functools.partial(
      pl.kernel,
      out_type=jax.ShapeDtypeStruct((_N_IDX, _DIM), jnp.float32),
      mesh=mesh,
      scratch_types=[
          pltpu.VMEM((_IDX_ROWS_PER_W, _IDXW), jnp.int32),
          pltpu.VMEM((_CHUNK, _DIM), jnp.float32),
          pltpu.SemaphoreType.DMA,
      ],
      compiler_params=pltpu.CompilerParams(use_tc_tiling_on_sc=False),
  )
  def gather_kernel(table_hbm, idx_hbm, out_hbm, idx_v, rows_v, sem):
    wid = lax.axis_index("s") * _NC + lax.axis_index("c")
    idx_row_base = wid * _IDX_ROWS_PER_W
    out_base = wid * _ROWS_PER_W

    pltpu.sync_copy(idx_hbm.at[pl.ds(idx_row_base, _IDX_ROWS_PER_W)], idx_v)

    def chunk_body(c, carry):
      copies = []
      for j in range(_GATHERS_PER_CHUNK):
        copies.append(
            pltpu.async_copy(
                table_hbm.at[idx_v.at[c * _GATHERS_PER_CHUNK + j]],
                rows_v.at[pl.ds(j * _IDXW, _IDXW)],
                sem,
            ))
      for cp in copies:
        cp.wait()
      pltpu.sync_copy(rows_v, out_hbm.at[pl.ds(out_base + c * _CHUNK, _CHUNK)])
      return carry

    lax.fori_loop(0, _N_CHUNKS, chunk_body, 0, unroll=False)

  return gather_kernel


_gather = _make_gather()


def kernel(input_, weight):
  idx = input_.reshape(-1).astype(jnp.int32).reshape(_N_IDX // _IDXW, _IDXW)
  out = _gather(weight, idx)
  return out.reshape(_B_ROWS, _SEQ, _DIM)


# speedup vs baseline: 1.0132x; 1.0132x over previous
import functools

import jax
import jax.numpy as jnp
from jax import lax
from jax.experimental import pallas as pl
from jax.experimental.pallas import tpu as pltpu
from jax.experimental.pallas import tpu_sc as plsc

_B_ROWS = 16384
_SEQ = 20
_DIM = 64
_N_IDX = _B_ROWS * _SEQ

_NC = 2
_NS = 16
_NW = _NC * _NS

_IDXW = 128
_ROWS_PER_W = _N_IDX // _NW
_IDX_ROWS_PER_W = _ROWS_PER_W // _IDXW

_GPC = 4
_CHUNK = _GPC * _IDXW
_N_CHUNKS = _ROWS_PER_W // _CHUNK
_NBUF = 2


def _make_gather():
  mesh = plsc.VectorSubcoreMesh(core_axis_name="c", subcore_axis_name="s")

  @functools.partial(
      pl.kernel,
      out_type=jax.ShapeDtypeStruct((_N_IDX, _DIM), jnp.float32),
      mesh=mesh,
      scratch_types=[
          pltpu.VMEM((_IDX_ROWS_PER_W, _IDXW), jnp.int32),
          pltpu.VMEM((_NBUF, _CHUNK, _DIM), jnp.float32),
          pltpu.SemaphoreType.DMA,
          pltpu.SemaphoreType.DMA,
          pltpu.SemaphoreType.DMA,
          pltpu.SemaphoreType.DMA,
      ],
      compiler_params=pltpu.CompilerParams(use_tc_tiling_on_sc=False),
  )
  def gather_kernel(table_hbm, idx_hbm, out_hbm, idx_v, rows_v,
                    sem_g0, sem_g1, sem_o0, sem_o1):
    sem_g = (sem_g0, sem_g1)
    sem_o = (sem_o0, sem_o1)
    wid = lax.axis_index("s") * _NC + lax.axis_index("c")
    idx_row_base = wid * _IDX_ROWS_PER_W
    out_base = wid * _ROWS_PER_W

    pltpu.sync_copy(idx_hbm.at[pl.ds(idx_row_base, _IDX_ROWS_PER_W)], idx_v)

    def g_copies(c, b):
      return [
          pltpu.make_async_copy(
              table_hbm.at[idx_v.at[c * _GPC + j]],
              rows_v.at[b].at[pl.ds(j * _IDXW, _IDXW)],
              sem_g[b],
          )
          for j in range(_GPC)
      ]

    def o_copy(c, b):
      return pltpu.make_async_copy(
          rows_v.at[b],
          out_hbm.at[pl.ds(out_base + c * _CHUNK, _CHUNK)],
          sem_o[b],
      )

    for b in range(_NBUF):
      for cp in g_copies(b, b):
        cp.start()

    def super_body(s, carry):
      c0 = s * _NBUF
      for b in range(_NBUF):
        c = c0 + b
        for cp in g_copies(c, b):
          cp.wait()
        o_copy(c, b).start()
        pb = (b - 1) % _NBUF
        cprev = c - 1
        nxt = cprev + _NBUF

        @pl.when(jnp.logical_and(cprev >= 0, nxt < _N_CHUNKS))
        def _():
          o_copy(cprev, pb).wait()
          for cp in g_copies(nxt, pb):
            cp.start()

      return carry

    lax.fori_loop(0, _N_CHUNKS // _NBUF, super_body, 0, unroll=False)

    for b in range(_NBUF):
      o_copy(_N_CHUNKS - _NBUF + b, b).wait()

  return gather_kernel


_gather = _make_gather()


def kernel(input_, weight):
  idx = input_.reshape(-1).astype(jnp.int32).reshape(_N_IDX // _IDXW, _IDXW)
  out = _gather(weight, idx)
  return out.reshape(_B_ROWS, _SEQ, _DIM)


# speedup vs baseline: 1.0175x; 1.0043x over previous
import functools

import jax
import jax.numpy as jnp
from jax import lax
from jax.experimental import pallas as pl
from jax.experimental.pallas import tpu as pltpu
from jax.experimental.pallas import tpu_sc as plsc

_B_ROWS = 16384
_SEQ = 20
_DIM = 64
_N_IDX = _B_ROWS * _SEQ

_NC = 2
_NS = 16
_NW = _NC * _NS

_IDXW = 128
_ROWS_PER_W = _N_IDX // _NW
_IDX_ROWS_PER_W = _ROWS_PER_W // _IDXW

_CHUNK = _IDXW
_N_CHUNKS = _ROWS_PER_W // _CHUNK
_NBUF = 10


def _make_gather():
  mesh = plsc.VectorSubcoreMesh(core_axis_name="c", subcore_axis_name="s")

  @functools.partial(
      pl.kernel,
      out_type=jax.ShapeDtypeStruct((_N_IDX, _DIM), jnp.float32),
      mesh=mesh,
      scratch_types=(
          [pltpu.VMEM((_IDX_ROWS_PER_W, _IDXW), jnp.int32),
           pltpu.VMEM((_NBUF, _CHUNK, _DIM), jnp.float32)]
          + [pltpu.SemaphoreType.DMA] * (2 * _NBUF)
      ),
      compiler_params=pltpu.CompilerParams(use_tc_tiling_on_sc=False),
  )
  def gather_kernel(table_hbm, idx_hbm, out_hbm, idx_v, rows_v, *sems):
    sem_g = sems[:_NBUF]
    sem_o = sems[_NBUF:]
    wid = lax.axis_index("s") * _NC + lax.axis_index("c")
    idx_row_base = wid * _IDX_ROWS_PER_W
    out_base = wid * _ROWS_PER_W

    pltpu.sync_copy(idx_hbm.at[pl.ds(idx_row_base, _IDX_ROWS_PER_W)], idx_v)

    def g_copy(c, b):
      return pltpu.make_async_copy(
          table_hbm.at[idx_v.at[c]], rows_v.at[b], sem_g[b])

    def o_copy(c, b):
      return pltpu.make_async_copy(
          rows_v.at[b],
          out_hbm.at[pl.ds(out_base + c * _CHUNK, _CHUNK)],
          sem_o[b],
      )

    for b in range(_NBUF):
      g_copy(b, b).start()

    def super_body(s, carry):
      c0 = s * _NBUF
      for b in range(_NBUF):
        c = c0 + b
        g_copy(c, b).wait()
        o_copy(c, b).start()
        pb = (b - 1) % _NBUF
        cprev = c - 1
        nxt = cprev + _NBUF

        @pl.when(jnp.logical_and(cprev >= 0, nxt < _N_CHUNKS))
        def _():
          o_copy(cprev, pb).wait()
          g_copy(nxt, pb).start()

      return carry

    lax.fori_loop(0, _N_CHUNKS // _NBUF, super_body, 0, unroll=False)

    for b in range(_NBUF):
      o_copy(_N_CHUNKS - _NBUF + b, b).wait()

  return gather_kernel


_gather = _make_gather()


def kernel(input_, weight):
  idx = input_.reshape(-1).astype(jnp.int32).reshape(_N_IDX // _IDXW, _IDXW)
  out = _gather(weight, idx)
  return out.reshape(_B_ROWS, _SEQ, _DIM)
